# baseline (speedup 1.0000x reference)
import functools

import jax
import jax.numpy as jnp
from jax import lax
from jax.experimental import pallas as pl
from jax.experimental.pallas import tpu as pltpu
from jax.experimental.pallas import tpu_sc as plsc

_N = 10000
_E = 320000
_F = 128
_C = 256
_G = 64

_NC = 2
_NS = 16
_K = 80

_RB = 400
_NRB = _N // _RB

_mesh = plsc.VectorSubcoreMesh(core_axis_name="c", subcore_axis_name="s")


@functools.partial(
    pl.kernel,
    out_type=jax.ShapeDtypeStruct((_NC, _N, 16), jnp.float32),
    mesh=_mesh,
    scratch_types=[
        pltpu.VMEM((_K, 16), jnp.float32),
        pltpu.VMEM((125, _K), jnp.int32),
        pltpu.VMEM_SHARED((10240, 16), jnp.float32),
    ],
    compiler_params=pltpu.CompilerParams(use_tc_tiling_on_sc=False),
)
def _sc_degree(dst_hbm, zeros_hbm, ones_hbm, degw_hbm, onesv, dstv, deg_sp):
    c = lax.axis_index("c")
    s = lax.axis_index("s")
    pltpu.sync_copy(zeros_hbm, deg_sp.at[pl.ds(s * 640, 640)])
    pltpu.sync_copy(ones_hbm, onesv)
    pltpu.sync_copy(dst_hbm.at[c, s], dstv)
    plsc.subcore_barrier()

    def body(j, carry):
        pltpu.sync_copy(onesv, deg_sp.at[dstv.at[j]], add=True)
        return carry

    lax.fori_loop(0, 125, body, 0)
    plsc.subcore_barrier()

    @pl.when(s == 0)
    def _():
        pltpu.sync_copy(deg_sp.at[pl.ds(0, _N)], degw_hbm.at[c])


_KS = 125
_NSL = 3
_NCH = _E // _NS // _KS


@functools.partial(
    pl.kernel,
    out_type=jax.ShapeDtypeStruct((_NC * _N, _F), jnp.float32),
    mesh=_mesh,
    scratch_types=[
        pltpu.VMEM((_NSL, _KS), jnp.int32),
        pltpu.VMEM((_NSL, 2, _KS), jnp.int32),
        pltpu.VMEM((_NSL, _KS, _F), jnp.float32),
        pltpu.VMEM_SHARED((_N, _F), jnp.float32),
        pltpu.SemaphoreType.DMA,
        pltpu.SemaphoreType.DMA,
        pltpu.SemaphoreType.DMA,
        pltpu.SemaphoreType.DMA,
        pltpu.SemaphoreType.DMA,
        pltpu.SemaphoreType.DMA,
        pltpu.SemaphoreType.DMA,
        pltpu.SemaphoreType.DMA,
    ],
    compiler_params=pltpu.CompilerParams(use_tc_tiling_on_sc=False),
)
def _sc_scatter(hp_hbm, src_hbm, dst_hbm, zrows_hbm, out_hbm,
                srcv, dstv, rows, acc_sp,
                gsem0, gsem1, gsem2, gsem3, ssem0, ssem1, ssem2, ssem3):
    c = lax.axis_index("c")
    s = lax.axis_index("s")
    gsem = (gsem0, gsem1, gsem2, gsem3)
    ssem = (ssem0, ssem1, ssem2, ssem3)
    rpt = _N // _NS
    for k in range(5):
        pltpu.async_copy(zrows_hbm, acc_sp.at[pl.ds(s * rpt + k * 125, 125)],
                         ssem0)

    def src_start(j, h):
        pltpu.async_copy(src_hbm.at[c, s, j], srcv.at[h], gsem[h])

    def src_wait(j, h):
        pltpu.make_async_copy(src_hbm.at[c, s, j], srcv.at[h],
                              gsem[h]).wait()

    def dst_start(j, h, p):
        pltpu.async_copy(dst_hbm.at[s, j], dstv.at[h, p], gsem[h])

    def dst_wait(j, h, p):
        pltpu.make_async_copy(dst_hbm.at[s, j], dstv.at[h, p],
                              gsem[h]).wait()

    def g_start(h):
        pltpu.async_copy(hp_hbm.at[srcv.at[h]], rows.at[h], gsem[h])

    def g_wait(h):
        pltpu.make_async_copy(hp_hbm.at[srcv.at[h]], rows.at[h],
                              gsem[h]).wait()

    def s_start(h, p):
        pltpu.async_copy(rows.at[h], acc_sp.at[dstv.at[h, p]], ssem[h],
                         add=True)

    def s_wait(h, p):
        pltpu.make_async_copy(rows.at[h], acc_sp.at[dstv.at[h, p]],
                              ssem[h]).wait()

    for k in range(5):
        pltpu.make_async_copy(zrows_hbm,
                              acc_sp.at[pl.ds(s * rpt + k * 125, 125)],
                              ssem0).wait()
    plsc.subcore_barrier()

    for h in range(_NSL):
        src_start(h, h)
        dst_start(h, h, 0)
        src_wait(h, h)
        dst_wait(h, h, 0)
        g_start(h)

    def outer(t, carry):
        p = lax.rem(t, 2)
        pn = 1 - p
        for h in range(_NSL):
            j = t * _NSL + h

            @pl.when(j < _NCH)
            def _():
                g_wait(h)

                @pl.when(j + _NSL < _NCH)
                def _():
                    src_start(j + _NSL, h)
                    dst_start(j + _NSL, h, pn)

                s_start(h, p)
                s_wait(h, p)

                @pl.when(j + _NSL < _NCH)
                def _():
                    src_wait(j + _NSL, h)
                    dst_wait(j + _NSL, h, pn)
                    g_start(h)

        return carry

    lax.fori_loop(0, (_NCH + _NSL - 1) // _NSL, outer, 0)
    plsc.subcore_barrier()
    pltpu.sync_copy(acc_sp.at[pl.ds(s * rpt, rpt)],
                    out_hbm.at[pl.ds(c * _N + s * rpt, rpt)])


def _tca1_body(x_ref, w_ref, b_ref, h_ref, starts_ref):
    i = pl.program_id(0)
    h_ref[...] = jnp.dot(x_ref[...], w_ref[...],
                         preferred_element_type=jnp.float32)

    @pl.when(i == 0)
    def _():
        starts_ref[...] = jnp.zeros_like(starts_ref)

    b = b_ref[...]
    g = lax.broadcasted_iota(jnp.int32, (1, 128), 1)
    starts_ref[...] += jnp.sum((b < g).astype(jnp.int32), axis=0,
                               keepdims=True)


def _tca1(x, w1, batch2d):
    return pl.pallas_call(
        _tca1_body,
        grid=(_NRB,),
        in_specs=[
            pl.BlockSpec((_RB, _F), lambda i: (i, 0)),
            pl.BlockSpec((_F, _C), lambda i: (0, 0)),
            pl.BlockSpec((_RB, 1), lambda i: (i, 0)),
        ],
        out_specs=[
            pl.BlockSpec((_RB, _C), lambda i: (i, 0)),
            pl.BlockSpec((1, 128), lambda i: (0, 0)),
        ],
        out_shape=[
            jax.ShapeDtypeStruct((_N, _C), jnp.float32),
            jax.ShapeDtypeStruct((1, 128), jnp.int32),
        ],
    )(x, w1, batch2d)


def _tca2_body(h_ref, deg_ref, hp_ref, dinv_ref):
    deg = deg_ref[0, :, 0:1] + deg_ref[1, :, 0:1] + 1.0
    dinv = lax.rsqrt(deg)
    h = h_ref[...]
    hp = h * dinv
    hp_ref[0] = hp[:, :_F]
    hp_ref[1] = hp[:, _F:]
    dinv_ref[...] = dinv


def _tca2(h, degw):
    return pl.pallas_call(
        _tca2_body,
        grid=(_NRB,),
        in_specs=[
            pl.BlockSpec((_RB, _C), lambda i: (i, 0)),
            pl.BlockSpec((_NC, _RB, 16), lambda i: (0, i, 0)),
        ],
        out_specs=[
            pl.BlockSpec((_NC, _RB, _F), lambda i: (0, i, 0)),
            pl.BlockSpec((_RB, 1), lambda i: (i, 0)),
        ],
        out_shape=[
            jax.ShapeDtypeStruct((_NC, _N, _F), jnp.float32),
            jax.ShapeDtypeStruct((_N, 1), jnp.float32),
        ],
    )(h, degw)


def _tcb_body(s_ref, hp_ref, dinv_ref, b1_ref, gamma_ref, beta_ref, w2_ref,
              x1b_ref, hp2_ref, x1pre_s, psum_s, psumsq_s):
    p = pl.program_id(0)
    i = pl.program_id(1)

    @pl.when(p == 0)
    def _():
        s = jnp.concatenate([s_ref[0], s_ref[1]], axis=1)
        hp = jnp.concatenate([hp_ref[0], hp_ref[1]], axis=1)
        x1 = dinv_ref[...] * (s + hp) + b1_ref[...]
        x1pre_s[pl.ds(i * _RB, _RB), :] = x1

        @pl.when(i == 0)
        def _():
            psum_s[...] = jnp.zeros_like(psum_s)
            psumsq_s[...] = jnp.zeros_like(psumsq_s)

        psum_s[...] += jnp.sum(x1, axis=0, keepdims=True)
        psumsq_s[...] += jnp.sum(x1 * x1, axis=0, keepdims=True)

    @pl.when(p == 1)
    def _():
        inv_n = 1.0 / _N
        mean = psum_s[...] * inv_n
        var = psumsq_s[...] * inv_n - mean * mean
        x1pre = x1pre_s[pl.ds(i * _RB, _RB), :]
        x1n = ((x1pre - mean) * lax.rsqrt(var + 1e-5)
               * gamma_ref[...] + beta_ref[...])
        x1r = jnp.maximum(x1n, 0.0)
        x1b_ref[...] = x1r
        h2 = jnp.dot(x1r, w2_ref[...], preferred_element_type=jnp.float32)
        hp2 = h2 * dinv_ref[...]
        hp2_ref[0] = hp2[:, :_F]
        hp2_ref[1] = hp2[:, _F:]


def _tcb(s1r, hp1r, dinv, b1r, gammar, betar, w2):
    return pl.pallas_call(
        _tcb_body,
        grid=(2, _NRB),
        in_specs=[
            pl.BlockSpec((_NC, _RB, _F), lambda p, i: (0, i * (1 - p), 0)),
            pl.BlockSpec((_NC, _RB, _F), lambda p, i: (0, i * (1 - p), 0)),
            pl.BlockSpec((_RB, 1), lambda p, i: (i, 0)),
            pl.BlockSpec((1, _C), lambda p, i: (0, 0)),
            pl.BlockSpec((1, _C), lambda p, i: (0, 0)),
            pl.BlockSpec((1, _C), lambda p, i: (0, 0)),
            pl.BlockSpec((_C, _C), lambda p, i: (0, 0)),
        ],
        out_specs=[
            pl.BlockSpec((_RB, _C), lambda p, i: (i * p, 0)),
            pl.BlockSpec((_NC, _RB, _F), lambda p, i: (0, i * p, 0)),
        ],
        out_shape=[
            jax.ShapeDtypeStruct((_N, _C), jnp.float32),
            jax.ShapeDtypeStruct((_NC, _N, _F), jnp.float32),
        ],
        scratch_shapes=[
            pltpu.VMEM((_N, _C), jnp.float32),
            pltpu.VMEM((1, _C), jnp.float32),
            pltpu.VMEM((1, _C), jnp.float32),
        ],
    )(s1r, hp1r, dinv, b1r, gammar, betar, w2)


def _tcd_body(starts_ref, x_ref, o_ref):
    g = pl.program_id(0)
    s = starts_ref[g]
    e = starts_ref[g + 1]
    k0 = lax.div(s, 8)
    k1 = lax.div(e + 7, 8)
    neg = jnp.float32(-jnp.inf)

    def body(k, carry):
        sm, mx = carry
        r0 = k * 8
        rid = r0 + lax.broadcasted_iota(jnp.int32, (8, _C), 0)
        m = (rid >= s) & (rid < e)
        a = x_ref[pl.ds(r0, 8), :]
        sm = sm + jnp.where(m, a, 0.0)
        mx = jnp.maximum(mx, jnp.where(m, a, neg))
        return sm, mx

    z = jnp.zeros((8, _C), jnp.float32)
    nf = jnp.full((8, _C), neg, jnp.float32)
    sm, mx = lax.fori_loop(k0, k1, body, (z, nf))
    cnt = jnp.maximum((e - s).astype(jnp.float32), 1.0)
    mean = jnp.sum(sm, axis=0, keepdims=True) / cnt
    mxr = jnp.max(mx, axis=0, keepdims=True)
    o_ref[0] = jnp.concatenate([mean, mxr], axis=1)


def _tcd(starts, xarr):
    grid_spec = pltpu.PrefetchScalarGridSpec(
        num_scalar_prefetch=1,
        grid=(_G,),
        in_specs=[
            pl.BlockSpec((_N, _C), lambda g, sref: (0, 0)),
        ],
        out_specs=pl.BlockSpec((1, 1, 2 * _C), lambda g, sref: (g, 0, 0)),
    )
    return pl.pallas_call(
        _tcd_body,
        grid_spec=grid_spec,
        out_shape=jax.ShapeDtypeStruct((_G, 1, 2 * _C), jnp.float32),
    )(starts, xarr).reshape(_G, 2 * _C)


def _tcc_body(s_ref, hp_ref, dinv_ref, b2_ref, x2_ref):
    s = jnp.concatenate([s_ref[0], s_ref[1]], axis=1)
    hp = jnp.concatenate([hp_ref[0], hp_ref[1]], axis=1)
    x2_ref[...] = jnp.maximum(dinv_ref[...] * (s + hp) + b2_ref[...], 0.0)


def _tcc(s2r, hp2r, dinv, b2r):
    return pl.pallas_call(
        _tcc_body,
        grid=(_NRB,),
        in_specs=[
            pl.BlockSpec((_NC, _RB, _F), lambda i: (0, i, 0)),
            pl.BlockSpec((_NC, _RB, _F), lambda i: (0, i, 0)),
            pl.BlockSpec((_RB, 1), lambda i: (i, 0)),
            pl.BlockSpec((1, _C), lambda i: (0, 0)),
        ],
        out_specs=pl.BlockSpec((_RB, _C), lambda i: (i, 0)),
        out_shape=jax.ShapeDtypeStruct((_N, _C), jnp.float32),
    )(s2r, hp2r, dinv, b2r)


def kernel(x, edge_index, batch, W1, b1, gamma, beta, W2, b2):
    x = x.astype(jnp.float32)
    src = edge_index[0]
    dst = edge_index[1]

    dst_deg = dst.reshape(_NC, _NS, 125, _K)
    srcr = src.reshape(_NS, _NCH, _KS)
    src_st = jnp.stack([srcr, srcr + _N])
    dst_st = dst.reshape(_NS, _NCH, _KS)

    zeros16 = jnp.zeros((640, 16), jnp.float32)
    ones16 = jnp.ones((_K, 16), jnp.float32)
    zrows = jnp.zeros((125, _F), jnp.float32)

    degw = _sc_degree(dst_deg, zeros16, ones16)

    batch2d = batch.reshape(_N, 1)
    h1, starts = _tca1(x, W1, batch2d)
    hp1r, dinv = _tca2(h1, degw)
    hp1 = hp1r.reshape(_NC * _N, _F)

    s1 = _sc_scatter(hp1, src_st, dst_st, zrows)

    s1r = s1.reshape(_NC, _N, _F)
    x1b, hp2r = _tcb(s1r, hp1r, dinv, b1.reshape(1, _C),
                     gamma.reshape(1, _C), beta.reshape(1, _C), W2)

    s2 = _sc_scatter(hp2r.reshape(_NC * _N, _F), src_st, dst_st, zrows)

    p1 = _tcd(starts.reshape(128), x1b)

    x2 = _tcc(s2.reshape(_NC, _N, _F), hp2r, dinv, b2.reshape(1, _C))
    p2 = _tcd(starts.reshape(128), x2)

    return jnp.concatenate([p1[:, :_C], p2[:, :_C],
                            p1[:, _C:], p2[:, _C:]], axis=1)


# speedup vs baseline: 1.0122x; 1.0122x over previous
import functools

import jax
import jax.numpy as jnp
from jax import lax
from jax.experimental import pallas as pl
from jax.experimental.pallas import tpu as pltpu
from jax.experimental.pallas import tpu_sc as plsc

_N = 10000
_E = 320000
_F = 128
_C = 256
_G = 64

_NC = 2
_NS = 16
_K = 80

_RB = 400
_NRB = _N // _RB

_mesh = plsc.VectorSubcoreMesh(core_axis_name="c", subcore_axis_name="s")


@functools.partial(
    pl.kernel,
    out_type=jax.ShapeDtypeStruct((_NC, _N, 16), jnp.float32),
    mesh=_mesh,
    scratch_types=[
        pltpu.VMEM((_K, 16), jnp.float32),
        pltpu.VMEM((125, _K), jnp.int32),
        pltpu.VMEM_SHARED((10240, 16), jnp.float32),
    ],
    compiler_params=pltpu.CompilerParams(use_tc_tiling_on_sc=False),
)
def _sc_degree(dst_hbm, zeros_hbm, ones_hbm, degw_hbm, onesv, dstv, deg_sp):
    c = lax.axis_index("c")
    s = lax.axis_index("s")
    pltpu.sync_copy(zeros_hbm, deg_sp.at[pl.ds(s * 640, 640)])
    pltpu.sync_copy(ones_hbm, onesv)
    pltpu.sync_copy(dst_hbm.at[c, s], dstv)
    plsc.subcore_barrier()

    def body(j, carry):
        pltpu.sync_copy(onesv, deg_sp.at[dstv.at[j]], add=True)
        return carry

    lax.fori_loop(0, 125, body, 0)
    plsc.subcore_barrier()

    @pl.when(s == 0)
    def _():
        pltpu.sync_copy(deg_sp.at[pl.ds(0, _N)], degw_hbm.at[c])


_KS = 125
_NSL = 3
_NCH = _E // _NS // _KS


@functools.partial(
    pl.kernel,
    out_type=jax.ShapeDtypeStruct((_NC * _N, _F), jnp.float32),
    mesh=_mesh,
    scratch_types=[
        pltpu.VMEM((_NSL, _KS), jnp.int32),
        pltpu.VMEM((_NSL, 2, _KS), jnp.int32),
        pltpu.VMEM((_NSL, _KS, _F), jnp.float32),
        pltpu.VMEM_SHARED((_N, _F), jnp.float32),
        pltpu.SemaphoreType.DMA,
        pltpu.SemaphoreType.DMA,
        pltpu.SemaphoreType.DMA,
        pltpu.SemaphoreType.DMA,
        pltpu.SemaphoreType.DMA,
        pltpu.SemaphoreType.DMA,
        pltpu.SemaphoreType.DMA,
        pltpu.SemaphoreType.DMA,
    ],
    compiler_params=pltpu.CompilerParams(use_tc_tiling_on_sc=False),
)
def _sc_scatter(hp_hbm, src_hbm, dst_hbm, zrows_hbm, out_hbm,
                srcv, dstv, rows, acc_sp,
                gsem0, gsem1, gsem2, gsem3, ssem0, ssem1, ssem2, ssem3):
    c = lax.axis_index("c")
    s = lax.axis_index("s")
    gsem = (gsem0, gsem1, gsem2, gsem3)
    ssem = (ssem0, ssem1, ssem2, ssem3)
    rpt = _N // _NS
    for k in range(5):
        pltpu.async_copy(zrows_hbm, acc_sp.at[pl.ds(s * rpt + k * 125, 125)],
                         ssem0)

    def src_start(j, h):
        pltpu.async_copy(src_hbm.at[c, s, j], srcv.at[h], gsem[h])

    def src_wait(j, h):
        pltpu.make_async_copy(src_hbm.at[c, s, j], srcv.at[h],
                              gsem[h]).wait()

    def dst_start(j, h, p):
        pltpu.async_copy(dst_hbm.at[s, j], dstv.at[h, p], gsem[h])

    def dst_wait(j, h, p):
        pltpu.make_async_copy(dst_hbm.at[s, j], dstv.at[h, p],
                              gsem[h]).wait()

    def g_start(h):
        pltpu.async_copy(hp_hbm.at[srcv.at[h]], rows.at[h], gsem[h])

    def g_wait(h):
        pltpu.make_async_copy(hp_hbm.at[srcv.at[h]], rows.at[h],
                              gsem[h]).wait()

    def s_start(h, p):
        pltpu.async_copy(rows.at[h], acc_sp.at[dstv.at[h, p]], ssem[h],
                         add=True)

    def s_wait(h, p):
        pltpu.make_async_copy(rows.at[h], acc_sp.at[dstv.at[h, p]],
                              ssem[h]).wait()

    for k in range(5):
        pltpu.make_async_copy(zrows_hbm,
                              acc_sp.at[pl.ds(s * rpt + k * 125, 125)],
                              ssem0).wait()
    plsc.subcore_barrier()

    for h in range(_NSL):
        src_start(h, h)
        dst_start(h, h, 0)
        src_wait(h, h)
        dst_wait(h, h, 0)
        g_start(h)

    def outer(t, carry):
        p = lax.rem(t, 2)
        pn = 1 - p
        for h in range(_NSL):
            j = t * _NSL + h

            @pl.when(j < _NCH)
            def _():
                g_wait(h)

                @pl.when(j + _NSL < _NCH)
                def _():
                    src_start(j + _NSL, h)
                    dst_start(j + _NSL, h, pn)

                s_start(h, p)
                s_wait(h, p)

                @pl.when(j + _NSL < _NCH)
                def _():
                    src_wait(j + _NSL, h)
                    dst_wait(j + _NSL, h, pn)
                    g_start(h)

        return carry

    lax.fori_loop(0, (_NCH + _NSL - 1) // _NSL, outer, 0)
    plsc.subcore_barrier()
    pltpu.sync_copy(acc_sp.at[pl.ds(s * rpt, rpt)],
                    out_hbm.at[pl.ds(c * _N + s * rpt, rpt)])


def _tca1_body(x_ref, w_ref, b_ref, h_ref, starts_ref):
    i = pl.program_id(0)
    h_ref[...] = jnp.dot(x_ref[...], w_ref[...],
                         preferred_element_type=jnp.float32)

    @pl.when(i == 0)
    def _():
        starts_ref[...] = jnp.zeros_like(starts_ref)

    b = b_ref[...]
    g = lax.broadcasted_iota(jnp.int32, (1, 128), 1)
    starts_ref[...] += jnp.sum((b < g).astype(jnp.int32), axis=0,
                               keepdims=True)


def _tca1(x, w1, batch2d):
    return pl.pallas_call(
        _tca1_body,
        grid=(_NRB,),
        in_specs=[
            pl.BlockSpec((_RB, _F), lambda i: (i, 0)),
            pl.BlockSpec((_F, _C), lambda i: (0, 0)),
            pl.BlockSpec((_RB, 1), lambda i: (i, 0)),
        ],
        out_specs=[
            pl.BlockSpec((_RB, _C), lambda i: (i, 0)),
            pl.BlockSpec((1, 128), lambda i: (0, 0)),
        ],
        out_shape=[
            jax.ShapeDtypeStruct((_N, _C), jnp.float32),
            jax.ShapeDtypeStruct((1, 128), jnp.int32),
        ],
    )(x, w1, batch2d)


def _tca2_body(h_ref, deg_ref, hp_ref, dinv_ref):
    deg = deg_ref[0, :, 0:1] + deg_ref[1, :, 0:1] + 1.0
    dinv = lax.rsqrt(deg)
    h = h_ref[...]
    hp = h * dinv
    hp_ref[0] = hp[:, :_F]
    hp_ref[1] = hp[:, _F:]
    dinv_ref[...] = dinv


def _tca2(h, degw):
    return pl.pallas_call(
        _tca2_body,
        grid=(_NRB,),
        in_specs=[
            pl.BlockSpec((_RB, _C), lambda i: (i, 0)),
            pl.BlockSpec((_NC, _RB, 16), lambda i: (0, i, 0)),
        ],
        out_specs=[
            pl.BlockSpec((_NC, _RB, _F), lambda i: (0, i, 0)),
            pl.BlockSpec((_RB, 1), lambda i: (i, 0)),
        ],
        out_shape=[
            jax.ShapeDtypeStruct((_NC, _N, _F), jnp.float32),
            jax.ShapeDtypeStruct((_N, 1), jnp.float32),
        ],
    )(h, degw)


def _tcb_body(s_ref, hp_ref, dinv_ref, b1_ref, gamma_ref, beta_ref, w2_ref,
              x1b_ref, hp2_ref, x1pre_s, psum_s, psumsq_s):
    p = pl.program_id(0)
    i = pl.program_id(1)

    @pl.when(p == 0)
    def _():
        s = jnp.concatenate([s_ref[0], s_ref[1]], axis=1)
        hp = jnp.concatenate([hp_ref[0], hp_ref[1]], axis=1)
        x1 = dinv_ref[...] * (s + hp) + b1_ref[...]
        x1pre_s[pl.ds(i * _RB, _RB), :] = x1

        @pl.when(i == 0)
        def _():
            psum_s[...] = jnp.zeros_like(psum_s)
            psumsq_s[...] = jnp.zeros_like(psumsq_s)

        psum_s[...] += jnp.sum(x1, axis=0, keepdims=True)
        psumsq_s[...] += jnp.sum(x1 * x1, axis=0, keepdims=True)

    @pl.when(p == 1)
    def _():
        inv_n = 1.0 / _N
        mean = psum_s[...] * inv_n
        var = psumsq_s[...] * inv_n - mean * mean
        x1pre = x1pre_s[pl.ds(i * _RB, _RB), :]
        x1n = ((x1pre - mean) * lax.rsqrt(var + 1e-5)
               * gamma_ref[...] + beta_ref[...])
        x1r = jnp.maximum(x1n, 0.0)
        x1b_ref[...] = x1r
        h2 = jnp.dot(x1r, w2_ref[...], preferred_element_type=jnp.float32)
        hp2 = h2 * dinv_ref[...]
        hp2_ref[0] = hp2[:, :_F]
        hp2_ref[1] = hp2[:, _F:]


def _tcb(s1r, hp1r, dinv, b1r, gammar, betar, w2):
    return pl.pallas_call(
        _tcb_body,
        grid=(2, _NRB),
        in_specs=[
            pl.BlockSpec((_NC, _RB, _F), lambda p, i: (0, i * (1 - p), 0)),
            pl.BlockSpec((_NC, _RB, _F), lambda p, i: (0, i * (1 - p), 0)),
            pl.BlockSpec((_RB, 1), lambda p, i: (i, 0)),
            pl.BlockSpec((1, _C), lambda p, i: (0, 0)),
            pl.BlockSpec((1, _C), lambda p, i: (0, 0)),
            pl.BlockSpec((1, _C), lambda p, i: (0, 0)),
            pl.BlockSpec((_C, _C), lambda p, i: (0, 0)),
        ],
        out_specs=[
            pl.BlockSpec((_RB, _C), lambda p, i: (i * p, 0)),
            pl.BlockSpec((_NC, _RB, _F), lambda p, i: (0, i * p, 0)),
        ],
        out_shape=[
            jax.ShapeDtypeStruct((_N, _C), jnp.float32),
            jax.ShapeDtypeStruct((_NC, _N, _F), jnp.float32),
        ],
        scratch_shapes=[
            pltpu.VMEM((_N, _C), jnp.float32),
            pltpu.VMEM((1, _C), jnp.float32),
            pltpu.VMEM((1, _C), jnp.float32),
        ],
    )(s1r, hp1r, dinv, b1r, gammar, betar, w2)


def _tcd_body(starts_ref, x1_ref, x2_ref, o_ref):
    g = pl.program_id(0)
    s = starts_ref[g]
    e = starts_ref[g + 1]
    k0 = lax.div(s, 8)
    k1 = lax.div(e + 7, 8)
    neg = jnp.float32(-jnp.inf)

    def body(k, carry):
        sm1, sm2, mx1, mx2 = carry
        r0 = k * 8
        rid = r0 + lax.broadcasted_iota(jnp.int32, (8, _C), 0)
        m = (rid >= s) & (rid < e)
        a1 = x1_ref[pl.ds(r0, 8), :]
        a2 = x2_ref[pl.ds(r0, 8), :]
        sm1 = sm1 + jnp.where(m, a1, 0.0)
        sm2 = sm2 + jnp.where(m, a2, 0.0)
        mx1 = jnp.maximum(mx1, jnp.where(m, a1, neg))
        mx2 = jnp.maximum(mx2, jnp.where(m, a2, neg))
        return sm1, sm2, mx1, mx2

    z = jnp.zeros((8, _C), jnp.float32)
    nf = jnp.full((8, _C), neg, jnp.float32)
    sm1, sm2, mx1, mx2 = lax.fori_loop(k0, k1, body, (z, z, nf, nf))
    cnt = jnp.maximum((e - s).astype(jnp.float32), 1.0)
    mean1 = jnp.sum(sm1, axis=0, keepdims=True) / cnt
    mean2 = jnp.sum(sm2, axis=0, keepdims=True) / cnt
    mxr1 = jnp.max(mx1, axis=0, keepdims=True)
    mxr2 = jnp.max(mx2, axis=0, keepdims=True)
    o_ref[0] = jnp.concatenate([mean1, mean2, mxr1, mxr2], axis=1)


def _tcd(starts, x1b, x2):
    grid_spec = pltpu.PrefetchScalarGridSpec(
        num_scalar_prefetch=1,
        grid=(_G,),
        in_specs=[
            pl.BlockSpec((_N, _C), lambda g, sref: (0, 0)),
            pl.BlockSpec((_N, _C), lambda g, sref: (0, 0)),
        ],
        out_specs=pl.BlockSpec((1, 1, 4 * _C), lambda g, sref: (g, 0, 0)),
    )
    return pl.pallas_call(
        _tcd_body,
        grid_spec=grid_spec,
        out_shape=jax.ShapeDtypeStruct((_G, 1, 4 * _C), jnp.float32),
    )(starts, x1b, x2).reshape(_G, 4 * _C)


def _tcc_body(s_ref, hp_ref, dinv_ref, b2_ref, x2_ref):
    s = jnp.concatenate([s_ref[0], s_ref[1]], axis=1)
    hp = jnp.concatenate([hp_ref[0], hp_ref[1]], axis=1)
    x2_ref[...] = jnp.maximum(dinv_ref[...] * (s + hp) + b2_ref[...], 0.0)


def _tcc(s2r, hp2r, dinv, b2r):
    return pl.pallas_call(
        _tcc_body,
        grid=(_NRB,),
        in_specs=[
            pl.BlockSpec((_NC, _RB, _F), lambda i: (0, i, 0)),
            pl.BlockSpec((_NC, _RB, _F), lambda i: (0, i, 0)),
            pl.BlockSpec((_RB, 1), lambda i: (i, 0)),
            pl.BlockSpec((1, _C), lambda i: (0, 0)),
        ],
        out_specs=pl.BlockSpec((_RB, _C), lambda i: (i, 0)),
        out_shape=jax.ShapeDtypeStruct((_N, _C), jnp.float32),
    )(s2r, hp2r, dinv, b2r)


def kernel(x, edge_index, batch, W1, b1, gamma, beta, W2, b2):
    x = x.astype(jnp.float32)
    src = edge_index[0]
    dst = edge_index[1]

    dst_deg = dst.reshape(_NC, _NS, 125, _K)
    srcr = src.reshape(_NS, _NCH, _KS)
    src_st = jnp.stack([srcr, srcr + _N])
    dst_st = dst.reshape(_NS, _NCH, _KS)

    zeros16 = jnp.zeros((640, 16), jnp.float32)
    ones16 = jnp.ones((_K, 16), jnp.float32)
    zrows = jnp.zeros((125, _F), jnp.float32)

    degw = _sc_degree(dst_deg, zeros16, ones16)

    batch2d = batch.reshape(_N, 1)
    h1, starts = _tca1(x, W1, batch2d)
    hp1r, dinv = _tca2(h1, degw)
    hp1 = hp1r.reshape(_NC * _N, _F)

    s1 = _sc_scatter(hp1, src_st, dst_st, zrows)

    s1r = s1.reshape(_NC, _N, _F)
    x1b, hp2r = _tcb(s1r, hp1r, dinv, b1.reshape(1, _C),
                     gamma.reshape(1, _C), beta.reshape(1, _C), W2)

    s2 = _sc_scatter(hp2r.reshape(_NC * _N, _F), src_st, dst_st, zrows)

    x2 = _tcc(s2.reshape(_NC, _N, _F), hp2r, dinv, b2.reshape(1, _C))

    return _tcd(starts.reshape(128), x1b, x2)
